# baseline (speedup 1.0000x reference)
import functools

import jax
import jax.numpy as jnp
import numpy as np
from jax import lax
from jax.experimental import pallas as pl
from jax.experimental.pallas import tpu as pltpu
from jax.experimental.pallas import tpu_sc as plsc

B = 128
SEQ = 20
S = 10
CH = 4
CDIM = 32
INDIM = 128
P = 16
VOCAB = 100000
N1 = SEQ * S
N2 = SEQ * S * S
T1 = B * N1
T2 = B * N2

ROWS_BLK = 2000


def _route_table_body(emb_ref, wl_ref, b_ref, ones_ref, out_ref):
    z = jnp.dot(emb_ref[...], wl_ref[...], preferred_element_type=jnp.float32)
    z = z + b_ref[...]
    nsq = jnp.dot(z * z, ones_ref[...], preferred_element_type=jnp.float32)
    out_ref[...] = z / jnp.maximum(jnp.sqrt(nsq), 1e-12)


def _route_table(emb, wlcat, bcat, blockones):
    grid = VOCAB // ROWS_BLK
    return pl.pallas_call(
        _route_table_body,
        grid=(grid,),
        in_specs=[
            pl.BlockSpec((ROWS_BLK, INDIM), lambda i: (i, 0)),
            pl.BlockSpec((INDIM, INDIM), lambda i: (0, 0)),
            pl.BlockSpec((1, INDIM), lambda i: (0, 0)),
            pl.BlockSpec((INDIM, INDIM), lambda i: (0, 0)),
        ],
        out_specs=pl.BlockSpec((ROWS_BLK, INDIM), lambda i: (i, 0)),
        out_shape=jax.ShapeDtypeStruct((VOCAB, INDIM), jnp.float32),
    )(emb, wlcat, bcat, blockones)


_NC = 2
_NS = 16
_NW = _NC * _NS
_CHUNK = 80
_PW1 = T1 // _NW
_PW2 = T2 // _NW
_NCH1 = _PW1 // _CHUNK
_NCH2 = _PW2 // _CHUNK


def _sc_gather(table, i1_flat, i2_flat):
    mesh = plsc.VectorSubcoreMesh(core_axis_name="c", subcore_axis_name="s")

    @functools.partial(
        pl.kernel,
        mesh=mesh,
        out_type=(
            jax.ShapeDtypeStruct((T1, INDIM), jnp.float32),
            jax.ShapeDtypeStruct((T2, INDIM), jnp.float32),
        ),
        scratch_types=[
            pltpu.VMEM((_CHUNK,), jnp.int32),
            pltpu.VMEM((_CHUNK, INDIM), jnp.float32),
            pltpu.SemaphoreType.DMA,
        ],
    )
    def k(table_hbm, i1_hbm, i2_hbm, out1_hbm, out2_hbm, idx_v, rows_v, sem):
        wid = lax.axis_index("s") * _NC + lax.axis_index("c")

        def mkbody(idx_hbm, out_hbm, wbase):
            def body(c, carry):
                base = wbase + c * _CHUNK
                pltpu.sync_copy(idx_hbm.at[pl.ds(base, _CHUNK)], idx_v)
                pltpu.async_copy(table_hbm.at[idx_v], rows_v, sem).wait()
                pltpu.sync_copy(rows_v, out_hbm.at[pl.ds(base, _CHUNK)])
                return carry
            return body

        lax.fori_loop(0, _NCH1, mkbody(i1_hbm, out1_hbm, wid * _PW1), 0)
        lax.fori_loop(0, _NCH2, mkbody(i2_hbm, out2_hbm, wid * _PW2), 0)

    return k(table, i1_flat, i2_flat)


def _main_body(h_ref, shl_ref, mask_ref, g1_ref, g1w_ref, g2w_ref,
               w0_ref, p0_ref, w1_ref, p1_ref,
               wlc_ref, bc_ref, bo_ref, pt_ref, e4_ref,
               w1c_ref, w1w_ref, w1p_ref, w2_ref, w3a_ref, w3b_ref,
               l1w_ref, l1b_ref, l2w_ref, l2b_ref, l3w_ref, out_ref):
    f32 = jnp.float32
    wlc = wlc_ref[...]
    bc = bc_ref[...]
    bo = bo_ref[...]

    def route(x):
        z = jnp.dot(x, wlc, preferred_element_type=f32) + bc
        nsq = jnp.dot(z * z, bo, preferred_element_type=f32)
        return z / jnp.maximum(jnp.sqrt(nsq), 1e-12)

    hb = h_ref[0]
    e0 = route(hb)
    item = route(shl_ref[0])
    msum = jnp.sum(mask_ref[0])
    srow = jnp.sum(item, axis=0, keepdims=True) / msum
    e4 = e4_ref[...]

    def agg(selfv, getnb, wv, pv, hop):
        nbs, logits = [], []
        for k in range(S):
            nb = getnb(k)
            m = nb * srow
            wt = (wv[:, k:k + 1] * w1w_ref[hop]
                  + jnp.dot(pv[:, k * P:(k + 1) * P], w1p_ref[hop],
                            preferred_element_type=f32))
            a = jnp.dot(m, w1c_ref[hop], preferred_element_type=f32) + wt
            a = jnp.where(a >= 0, a, 0.2 * a)
            nbs.append(nb)
            logits.append(jnp.dot(a, w2_ref[hop], preferred_element_type=f32))
        mx = logits[0]
        for k in range(1, S):
            mx = jnp.maximum(mx, logits[k])
        ex = [jnp.exp(l - mx) for l in logits]
        den = ex[0]
        for k in range(1, S):
            den = den + ex[k]
        pooled = None
        for k in range(S):
            alpha = jnp.dot(ex[k] / den, e4, preferred_element_type=f32)
            term = alpha * nbs[k]
            pooled = term if pooled is None else pooled + term
        o = (jnp.dot(selfv, w3a_ref[hop], preferred_element_type=f32)
             + jnp.dot(pooled, w3b_ref[hop], preferred_element_type=f32))
        return jnp.maximum(o, 0.0)

    g1 = g1_ref[0]
    g1w = g1w_ref[0]
    g2w = g2w_ref[0]

    h1 = agg(g1, lambda k: g2w[:, k * INDIM:(k + 1) * INDIM],
             w1_ref[0], p1_ref[0], 0)
    h0 = agg(e0, lambda k: g1w[:, k * INDIM:(k + 1) * INDIM],
             w0_ref[0], p0_ref[0], 0)
    pt = pt_ref[...]
    fin = agg(h0,
              lambda k: jnp.dot(pt[k * SEQ:(k + 1) * SEQ], h1,
                                preferred_element_type=f32),
              w0_ref[0], p0_ref[0], 1)

    q = (jnp.dot(fin, l1w_ref[...], preferred_element_type=f32) + l1b_ref[...]
         + jnp.dot(hb, l2w_ref[...], preferred_element_type=f32) + l2b_ref[...])
    alpha = jnp.dot(jax.nn.sigmoid(q), l3w_ref[...], preferred_element_type=f32)
    out_ref[0] = alpha * hb + (1.0 - alpha) * fin


def _main(h, shl, mask3, g1, g1w, g2w, w0r, p0r, w1r, p1r,
          wlcat, bcat, blockones, pt, e4,
          bdw1c, w1w, w1p, bdw2, bdw3a, bdw3b, l1w, l1b, l2w, l2b, l3w):
    full = lambda shape: pl.BlockSpec(shape, lambda b: (0,) * len(shape))
    batch = lambda shape: pl.BlockSpec((1,) + shape, lambda b: (b,) + (0,) * len(shape))
    return pl.pallas_call(
        _main_body,
        grid=(B,),
        in_specs=[
            batch((SEQ, INDIM)),
            batch((SEQ, INDIM)),
            batch((1, SEQ)),
            batch((N1, INDIM)),
            batch((SEQ, S * INDIM)),
            batch((N1, S * INDIM)),
            batch((SEQ, S)),
            batch((SEQ, S * P)),
            batch((N1, S)),
            batch((N1, S * P)),
            full((INDIM, INDIM)),
            full((1, INDIM)),
            full((INDIM, INDIM)),
            full((N1, N1)),
            full((CH, INDIM)),
            full((2, INDIM, INDIM)),
            full((2, 1, INDIM)),
            full((2, P, INDIM)),
            full((2, INDIM, CH)),
            full((2, INDIM, INDIM)),
            full((2, INDIM, INDIM)),
            full((INDIM, INDIM)),
            full((1, INDIM)),
            full((INDIM, INDIM)),
            full((1, INDIM)),
            full((INDIM, 1)),
        ],
        out_specs=pl.BlockSpec((1, SEQ, INDIM), lambda b: (b, 0, 0)),
        out_shape=jax.ShapeDtypeStruct((B, SEQ, INDIM), jnp.float32),
    )(h, shl, mask3, g1, g1w, g2w, w0r, p0r, w1r, p1r,
      wlcat, bcat, blockones, pt, e4,
      bdw1c, w1w, w1p, bdw2, bdw3a, bdw3b, l1w, l1b, l2w, l2b, l3w)


_PT_NP = np.zeros((N1, N1), np.float32)
_r = np.arange(N1)
_PT_NP[_r, (_r % SEQ) * S + (_r // SEQ)] = 1.0


def kernel(h, item_neighbors_0_0, item_neighbors_1_0, item_neighbors_2_0,
           weight_neighbors_0_0, weight_neighbors_1_0, pos_neighbors_0_0,
           pos_neighbors_1_0, pos_before, pos_after, seq_hidden_local,
           mask_item, embedding, weight_list, bias_list, agg_W1, agg_W2,
           agg_W3, lin1_W, lin1_b, lin2_W, lin2_b, lin3_W):
    f32 = jnp.float32
    eye4 = jnp.eye(CH, dtype=f32)

    wlcat = jnp.concatenate([weight_list[j] for j in range(CH)], axis=1)
    bcat = jnp.concatenate([bias_list[j] for j in range(CH)], axis=1)
    blockones = jnp.kron(eye4, jnp.ones((CDIM, CDIM), f32))
    e4 = jnp.kron(eye4, jnp.ones((1, CDIM), f32))
    bdw1c = jnp.stack([jnp.kron(eye4, agg_W1[i, :CDIM]) for i in range(2)])
    w1w = jnp.stack([jnp.tile(agg_W1[i, CDIM:CDIM + 1], (1, CH)) for i in range(2)])
    w1p = jnp.stack([jnp.tile(agg_W1[i, CDIM + 1:], (1, CH)) for i in range(2)])
    bdw2 = jnp.stack([jnp.kron(eye4, agg_W2[i]) for i in range(2)])
    bdw3a = jnp.stack([jnp.kron(eye4, agg_W3[i, :CDIM]) for i in range(2)])
    bdw3b = jnp.stack([jnp.kron(eye4, agg_W3[i, CDIM:]) for i in range(2)])
    pt = jnp.asarray(_PT_NP)

    i1_flat = item_neighbors_1_0.astype(jnp.int32).reshape(T1)
    i2_flat = item_neighbors_2_0.astype(jnp.int32).reshape(T2)
    w0r = weight_neighbors_0_0.reshape(B, SEQ, S)
    w1r = weight_neighbors_1_0.reshape(B, N1, S)
    p0r = pos_neighbors_0_0.reshape(B, SEQ, S * P)
    p1r = pos_neighbors_1_0.reshape(B, N1, S * P)
    mask3 = mask_item.reshape(B, 1, SEQ)
    l1b = lin1_b.reshape(1, INDIM)
    l2b = lin2_b.reshape(1, INDIM)

    table = _route_table(embedding, wlcat, bcat, blockones)
    g1f, g2f = _sc_gather(table, i1_flat, i2_flat)
    g1 = g1f.reshape(B, N1, INDIM)
    g1wide = g1f.reshape(B, SEQ, S * INDIM)
    g2wide = g2f.reshape(B, N1, S * INDIM)
    return _main(h, seq_hidden_local, mask3, g1, g1wide, g2wide,
                 w0r, p0r, w1r, p1r,
                 wlcat, bcat, blockones, pt, e4,
                 bdw1c, w1w, w1p, bdw2, bdw3a, bdw3b,
                 lin1_W, l1b, lin2_W, l2b, lin3_W)


# speedup vs baseline: 1.9357x; 1.9357x over previous
import functools

import jax
import jax.numpy as jnp
import numpy as np
from jax import lax
from jax.experimental import pallas as pl
from jax.experimental.pallas import tpu as pltpu
from jax.experimental.pallas import tpu_sc as plsc

B = 128
SEQ = 20
S = 10
CH = 4
CDIM = 32
INDIM = 128
P = 16
VOCAB = 100000
N1 = SEQ * S
N2 = SEQ * S * S
T1 = B * N1
T2 = B * N2

ROWS_BLK = 2000


def _route_table_body(emb_ref, wl_ref, b_ref, ones_ref, out_ref):
    z = jnp.dot(emb_ref[...], wl_ref[...], preferred_element_type=jnp.float32)
    z = z + b_ref[...]
    nsq = jnp.dot(z * z, ones_ref[...], preferred_element_type=jnp.float32)
    out_ref[...] = z / jnp.maximum(jnp.sqrt(nsq), 1e-12)


def _route_table(emb, wlcat, bcat, blockones):
    grid = VOCAB // ROWS_BLK
    return pl.pallas_call(
        _route_table_body,
        grid=(grid,),
        in_specs=[
            pl.BlockSpec((ROWS_BLK, INDIM), lambda i: (i, 0)),
            pl.BlockSpec((INDIM, INDIM), lambda i: (0, 0)),
            pl.BlockSpec((1, INDIM), lambda i: (0, 0)),
            pl.BlockSpec((INDIM, INDIM), lambda i: (0, 0)),
        ],
        out_specs=pl.BlockSpec((ROWS_BLK, INDIM), lambda i: (i, 0)),
        out_shape=jax.ShapeDtypeStruct((VOCAB, INDIM), jnp.float32),
    )(emb, wlcat, bcat, blockones)


_NC = 2
_NS = 16
_NW = _NC * _NS
_CHUNK = 80
_PW1 = T1 // _NW
_PW2 = T2 // _NW
_NCH1 = _PW1 // _CHUNK
_NCH2 = _PW2 // _CHUNK


def _sc_gather(table, i1_flat, i2_flat):
    mesh = plsc.VectorSubcoreMesh(core_axis_name="c", subcore_axis_name="s")

    @functools.partial(
        pl.kernel,
        mesh=mesh,
        out_type=(
            jax.ShapeDtypeStruct((T1, INDIM), jnp.float32),
            jax.ShapeDtypeStruct((T2, INDIM), jnp.float32),
        ),
        scratch_types=[
            pltpu.VMEM((_CHUNK,), jnp.int32),
            pltpu.VMEM((_CHUNK, INDIM), jnp.float32),
            pltpu.SemaphoreType.DMA,
        ],
    )
    def k(table_hbm, i1_hbm, i2_hbm, out1_hbm, out2_hbm, idx_v, rows_v, sem):
        wid = lax.axis_index("s") * _NC + lax.axis_index("c")

        def mkbody(idx_hbm, out_hbm, wbase):
            def body(c, carry):
                base = wbase + c * _CHUNK
                pltpu.sync_copy(idx_hbm.at[pl.ds(base, _CHUNK)], idx_v)
                pltpu.async_copy(table_hbm.at[idx_v], rows_v, sem).wait()
                pltpu.sync_copy(rows_v, out_hbm.at[pl.ds(base, _CHUNK)])
                return carry
            return body

        lax.fori_loop(0, _NCH1, mkbody(i1_hbm, out1_hbm, wid * _PW1), 0)
        lax.fori_loop(0, _NCH2, mkbody(i2_hbm, out2_hbm, wid * _PW2), 0)

    return k(table, i1_flat, i2_flat)


NB = 8
GRID3 = B // NB


def _main_body(h_ref, shl_ref, mask_ref, g1_ref, g1w_ref, g2w_ref,
               w0_ref, p0_ref, w1_ref, p1_ref,
               wlc_ref, bc_ref, bo_ref, pt_ref, e4_ref,
               w1c_ref, w1w_ref, w1p_ref, w2_ref, w3a_ref, w3b_ref,
               l1w_ref, l1b_ref, l2w_ref, l2b_ref, l3w_ref,
               seg_ref, exp200_ref, exp20_ref, out_ref):
    f32 = jnp.float32
    wlc = wlc_ref[...]
    bc = bc_ref[...]
    bo = bo_ref[...]

    def route(x):
        z = jnp.dot(x, wlc, preferred_element_type=f32) + bc
        nsq = jnp.dot(z * z, bo, preferred_element_type=f32)
        return z / jnp.maximum(jnp.sqrt(nsq), 1e-12)

    hf = h_ref[...].reshape(NB * SEQ, INDIM)
    e0 = route(hf)
    item = route(shl_ref[...].reshape(NB * SEQ, INDIM))
    msum = jnp.sum(mask_ref[...], axis=2)
    srow = jnp.dot(seg_ref[...], item, preferred_element_type=f32) / msum
    s1600 = jnp.dot(exp200_ref[...], srow, preferred_element_type=f32)
    s160 = jnp.dot(exp20_ref[...], srow, preferred_element_type=f32)
    e4 = e4_ref[...]

    def agg(selfv, getnb, wv3, pv3, sfull, hop, n):
        rows = NB * n
        nbs, logits = [], []
        for k in range(S):
            nb = getnb(k)
            m = nb * sfull
            wt = (wv3[:, :, k:k + 1].reshape(rows, 1) * w1w_ref[hop]
                  + jnp.dot(pv3[:, :, k * P:(k + 1) * P].reshape(rows, P),
                            w1p_ref[hop], preferred_element_type=f32))
            a = jnp.dot(m, w1c_ref[hop], preferred_element_type=f32) + wt
            a = jnp.where(a >= 0, a, 0.2 * a)
            nbs.append(nb)
            logits.append(jnp.dot(a, w2_ref[hop], preferred_element_type=f32))
        mx = logits[0]
        for k in range(1, S):
            mx = jnp.maximum(mx, logits[k])
        ex = [jnp.exp(l - mx) for l in logits]
        den = ex[0]
        for k in range(1, S):
            den = den + ex[k]
        pooled = None
        for k in range(S):
            alpha = jnp.dot(ex[k] / den, e4, preferred_element_type=f32)
            term = alpha * nbs[k]
            pooled = term if pooled is None else pooled + term
        o = (jnp.dot(selfv, w3a_ref[hop], preferred_element_type=f32)
             + jnp.dot(pooled, w3b_ref[hop], preferred_element_type=f32))
        return jnp.maximum(o, 0.0)

    g1f = g1_ref[...].reshape(NB * N1, INDIM)
    g1w3 = g1w_ref[...]
    g2w3 = g2w_ref[...]

    h1 = agg(g1f,
             lambda k: g2w3[:, :, k * INDIM:(k + 1) * INDIM].reshape(NB * N1, INDIM),
             w1_ref[...], p1_ref[...], s1600, 0, N1)
    h0 = agg(e0,
             lambda k: g1w3[:, :, k * INDIM:(k + 1) * INDIM].reshape(NB * SEQ, INDIM),
             w0_ref[...], p0_ref[...], s160, 0, SEQ)
    pt = pt_ref[...]
    h1_3 = h1.reshape(NB, N1, INDIM)
    nbf = jnp.stack([jnp.dot(pt, h1_3[b], preferred_element_type=f32)
                     for b in range(NB)])
    fin = agg(h0,
              lambda k: nbf[:, k * SEQ:(k + 1) * SEQ, :].reshape(NB * SEQ, INDIM),
              w0_ref[...], p0_ref[...], s160, 1, SEQ)

    q = (jnp.dot(fin, l1w_ref[...], preferred_element_type=f32) + l1b_ref[...]
         + jnp.dot(hf, l2w_ref[...], preferred_element_type=f32) + l2b_ref[...])
    alpha = jnp.dot(jax.nn.sigmoid(q), l3w_ref[...], preferred_element_type=f32)
    out_ref[...] = (alpha * hf + (1.0 - alpha) * fin).reshape(NB, SEQ, INDIM)


def _main(h, shl, mask3, g1, g1w, g2w, w0r, p0r, w1r, p1r,
          wlcat, bcat, blockones, pt, e4,
          bdw1c, w1w, w1p, bdw2, bdw3a, bdw3b, l1w, l1b, l2w, l2b, l3w,
          seg, exp200, exp20):
    full = lambda shape: pl.BlockSpec(shape, lambda b: (0,) * len(shape))
    batch = lambda shape: pl.BlockSpec((NB,) + shape, lambda b: (b,) + (0,) * len(shape))
    return pl.pallas_call(
        _main_body,
        grid=(GRID3,),
        in_specs=[
            batch((SEQ, INDIM)),
            batch((SEQ, INDIM)),
            batch((1, SEQ)),
            batch((N1, INDIM)),
            batch((SEQ, S * INDIM)),
            batch((N1, S * INDIM)),
            batch((SEQ, S)),
            batch((SEQ, S * P)),
            batch((N1, S)),
            batch((N1, S * P)),
            full((INDIM, INDIM)),
            full((1, INDIM)),
            full((INDIM, INDIM)),
            full((N1, N1)),
            full((CH, INDIM)),
            full((2, INDIM, INDIM)),
            full((2, 1, INDIM)),
            full((2, P, INDIM)),
            full((2, INDIM, CH)),
            full((2, INDIM, INDIM)),
            full((2, INDIM, INDIM)),
            full((INDIM, INDIM)),
            full((1, INDIM)),
            full((INDIM, INDIM)),
            full((1, INDIM)),
            full((INDIM, 1)),
            full((NB, NB * SEQ)),
            full((NB * N1, NB)),
            full((NB * SEQ, NB)),
        ],
        out_specs=pl.BlockSpec((NB, SEQ, INDIM), lambda b: (b, 0, 0)),
        out_shape=jax.ShapeDtypeStruct((B, SEQ, INDIM), jnp.float32),
    )(h, shl, mask3, g1, g1w, g2w, w0r, p0r, w1r, p1r,
      wlcat, bcat, blockones, pt, e4,
      bdw1c, w1w, w1p, bdw2, bdw3a, bdw3b, l1w, l1b, l2w, l2b, l3w,
      seg, exp200, exp20)


_PT_NP = np.zeros((N1, N1), np.float32)
_r = np.arange(N1)
_PT_NP[_r, (_r % SEQ) * S + (_r // SEQ)] = 1.0

_SEG_NP = np.zeros((NB, NB * SEQ), np.float32)
_SEG_NP[np.arange(NB * SEQ) // SEQ, np.arange(NB * SEQ)] = 1.0
_EXP200_NP = np.zeros((NB * N1, NB), np.float32)
_EXP200_NP[np.arange(NB * N1), np.arange(NB * N1) // N1] = 1.0
_EXP20_NP = np.zeros((NB * SEQ, NB), np.float32)
_EXP20_NP[np.arange(NB * SEQ), np.arange(NB * SEQ) // SEQ] = 1.0


def kernel(h, item_neighbors_0_0, item_neighbors_1_0, item_neighbors_2_0,
           weight_neighbors_0_0, weight_neighbors_1_0, pos_neighbors_0_0,
           pos_neighbors_1_0, pos_before, pos_after, seq_hidden_local,
           mask_item, embedding, weight_list, bias_list, agg_W1, agg_W2,
           agg_W3, lin1_W, lin1_b, lin2_W, lin2_b, lin3_W):
    f32 = jnp.float32
    eye4 = jnp.eye(CH, dtype=f32)

    wlcat = jnp.concatenate([weight_list[j] for j in range(CH)], axis=1)
    bcat = jnp.concatenate([bias_list[j] for j in range(CH)], axis=1)
    blockones = jnp.kron(eye4, jnp.ones((CDIM, CDIM), f32))
    e4 = jnp.kron(eye4, jnp.ones((1, CDIM), f32))
    bdw1c = jnp.stack([jnp.kron(eye4, agg_W1[i, :CDIM]) for i in range(2)])
    w1w = jnp.stack([jnp.tile(agg_W1[i, CDIM:CDIM + 1], (1, CH)) for i in range(2)])
    w1p = jnp.stack([jnp.tile(agg_W1[i, CDIM + 1:], (1, CH)) for i in range(2)])
    bdw2 = jnp.stack([jnp.kron(eye4, agg_W2[i]) for i in range(2)])
    bdw3a = jnp.stack([jnp.kron(eye4, agg_W3[i, :CDIM]) for i in range(2)])
    bdw3b = jnp.stack([jnp.kron(eye4, agg_W3[i, CDIM:]) for i in range(2)])
    pt = jnp.asarray(_PT_NP)
    seg = jnp.asarray(_SEG_NP)
    exp200 = jnp.asarray(_EXP200_NP)
    exp20 = jnp.asarray(_EXP20_NP)

    i1_flat = item_neighbors_1_0.astype(jnp.int32).reshape(T1)
    i2_flat = item_neighbors_2_0.astype(jnp.int32).reshape(T2)
    w0r = weight_neighbors_0_0.reshape(B, SEQ, S)
    w1r = weight_neighbors_1_0.reshape(B, N1, S)
    p0r = pos_neighbors_0_0.reshape(B, SEQ, S * P)
    p1r = pos_neighbors_1_0.reshape(B, N1, S * P)
    mask3 = mask_item.reshape(B, 1, SEQ)
    l1b = lin1_b.reshape(1, INDIM)
    l2b = lin2_b.reshape(1, INDIM)

    table = _route_table(embedding, wlcat, bcat, blockones)
    g1f, g2f = _sc_gather(table, i1_flat, i2_flat)
    g1 = g1f.reshape(B, N1, INDIM)
    g1wide = g1f.reshape(B, SEQ, S * INDIM)
    g2wide = g2f.reshape(B, N1, S * INDIM)
    return _main(h, seq_hidden_local, mask3, g1, g1wide, g2wide,
                 w0r, p0r, w1r, p1r,
                 wlcat, bcat, blockones, pt, e4,
                 bdw1c, w1w, w1p, bdw2, bdw3a, bdw3b,
                 lin1_W, l1b, lin2_W, l2b, lin3_W,
                 seg, exp200, exp20)


# speedup vs baseline: 2.3346x; 1.2061x over previous
import functools

import jax
import jax.numpy as jnp
import numpy as np
from jax import lax
from jax.experimental import pallas as pl
from jax.experimental.pallas import tpu as pltpu
from jax.experimental.pallas import tpu_sc as plsc

B = 128
SEQ = 20
S = 10
CH = 4
CDIM = 32
INDIM = 128
P = 16
VOCAB = 100000
N1 = SEQ * S
N2 = SEQ * S * S
T1 = B * N1
T2 = B * N2

ROWS_BLK = 2000


def _route_table_body(emb_ref, wl_ref, b_ref, ones_ref, out_ref):
    z = jnp.dot(emb_ref[...], wl_ref[...], preferred_element_type=jnp.float32)
    z = z + b_ref[...]
    nsq = jnp.dot(z * z, ones_ref[...], preferred_element_type=jnp.float32)
    out_ref[...] = z / jnp.maximum(jnp.sqrt(nsq), 1e-12)


def _route_table(emb, wlcat, bcat, blockones):
    grid = VOCAB // ROWS_BLK
    return pl.pallas_call(
        _route_table_body,
        grid=(grid,),
        in_specs=[
            pl.BlockSpec((ROWS_BLK, INDIM), lambda i: (i, 0)),
            pl.BlockSpec((INDIM, INDIM), lambda i: (0, 0)),
            pl.BlockSpec((1, INDIM), lambda i: (0, 0)),
            pl.BlockSpec((INDIM, INDIM), lambda i: (0, 0)),
        ],
        out_specs=pl.BlockSpec((ROWS_BLK, INDIM), lambda i: (i, 0)),
        out_shape=jax.ShapeDtypeStruct((VOCAB, INDIM), jnp.float32),
    )(emb, wlcat, bcat, blockones)


_NC = 2
_NS = 16
_NW = _NC * _NS
_WCH = 80
_NPC = 5
_WAVE = _WCH * _NPC


def _sc_gather(table, i1h, i2h, t1, t2):
    pw1, pw2 = t1 // _NW, t2 // _NW
    nw2 = pw2 // _WAVE
    mesh = plsc.VectorSubcoreMesh(core_axis_name="c", subcore_axis_name="s")

    @functools.partial(
        pl.kernel,
        mesh=mesh,
        out_type=(
            jax.ShapeDtypeStruct((t1, INDIM), jnp.float32),
            jax.ShapeDtypeStruct((t2, INDIM), jnp.float32),
        ),
        scratch_types=[
            pltpu.VMEM((pw1,), jnp.int32),
            pltpu.VMEM((pw2,), jnp.int32),
            pltpu.VMEM((_WAVE, INDIM), jnp.float32),
            pltpu.VMEM((_WAVE, INDIM), jnp.float32),
            pltpu.SemaphoreType.DMA,
            pltpu.SemaphoreType.DMA,
        ],
    )
    def k(table_hbm, i1_hbm, i2_hbm, out1_hbm, out2_hbm,
          idx1_v, idx2_v, ra, rb, sema, semb):
        wid = lax.axis_index("s") * _NC + lax.axis_index("c")
        b1 = wid * pw1
        b2 = wid * pw2
        pltpu.sync_copy(i1_hbm.at[pl.ds(b1, pw1)], idx1_v)
        pltpu.sync_copy(i2_hbm.at[pl.ds(b2, pw2)], idx2_v)

        def fire(idx_v, woff, buf, sem):
            for c in range(_NPC):
                pltpu.async_copy(
                    table_hbm.at[idx_v.at[pl.ds(woff + c * _WCH, _WCH)]],
                    buf.at[pl.ds(c * _WCH, _WCH)], sem)

        def drain(buf, sem):
            pltpu.make_async_copy(table_hbm.at[pl.ds(0, _WAVE)], buf, sem).wait()

        fire(idx1_v, 0, ra, sema)
        drain(ra, sema)
        pltpu.sync_copy(ra, out1_hbm.at[pl.ds(b1, pw1)])

        fire(idx2_v, 0, ra, sema)

        def body(i, carry):
            w0 = 2 * i
            fire(idx2_v, (w0 + 1) * _WAVE, rb, semb)
            drain(ra, sema)
            pltpu.sync_copy(ra, out2_hbm.at[pl.ds(b2 + w0 * _WAVE, _WAVE)])

            @pl.when(i < nw2 // 2 - 1)
            def _():
                fire(idx2_v, (w0 + 2) * _WAVE, ra, sema)

            drain(rb, semb)
            pltpu.sync_copy(rb, out2_hbm.at[pl.ds(b2 + (w0 + 1) * _WAVE, _WAVE)])
            return carry

        lax.fori_loop(0, nw2 // 2, body, 0)

    return k(table, i1h, i2h)


NB = 8
GRID3 = B // NB


def _main_body(h_ref, shl_ref, mask_ref, g1_ref, g1w_ref, g2w_ref,
               w0_ref, p0_ref, w1_ref, p1_ref,
               wlc_ref, bc_ref, bo_ref, pt_ref, e4_ref,
               w1c_ref, w1w_ref, w1p_ref, w2_ref, w3a_ref, w3b_ref,
               l1w_ref, l1b_ref, l2w_ref, l2b_ref, l3w_ref,
               seg_ref, exp200_ref, exp20_ref, out_ref):
    f32 = jnp.float32
    wlc = wlc_ref[...]
    bc = bc_ref[...]
    bo = bo_ref[...]

    def route(x):
        z = jnp.dot(x, wlc, preferred_element_type=f32) + bc
        nsq = jnp.dot(z * z, bo, preferred_element_type=f32)
        return z / jnp.maximum(jnp.sqrt(nsq), 1e-12)

    hf = h_ref[...].reshape(NB * SEQ, INDIM)
    e0 = route(hf)
    item = route(shl_ref[...].reshape(NB * SEQ, INDIM))
    msum = jnp.sum(mask_ref[...], axis=2)
    srow = jnp.dot(seg_ref[...], item, preferred_element_type=f32) / msum
    s1600 = jnp.dot(exp200_ref[...], srow, preferred_element_type=f32)
    s160 = jnp.dot(exp20_ref[...], srow, preferred_element_type=f32)
    e4 = e4_ref[...]

    def agg(selfv, getnb, wv3, pv3, sfull, hop, n):
        rows = NB * n
        nbs, logits = [], []
        for k in range(S):
            nb = getnb(k)
            m = nb * sfull
            wt = (wv3[:, :, k:k + 1].reshape(rows, 1) * w1w_ref[hop]
                  + jnp.dot(pv3[:, :, k * P:(k + 1) * P].reshape(rows, P),
                            w1p_ref[hop], preferred_element_type=f32))
            a = jnp.dot(m, w1c_ref[hop], preferred_element_type=f32) + wt
            a = jnp.where(a >= 0, a, 0.2 * a)
            nbs.append(nb)
            logits.append(jnp.dot(a, w2_ref[hop], preferred_element_type=f32))
        mx = logits[0]
        for k in range(1, S):
            mx = jnp.maximum(mx, logits[k])
        ex = [jnp.exp(l - mx) for l in logits]
        den = ex[0]
        for k in range(1, S):
            den = den + ex[k]
        pooled = None
        for k in range(S):
            alpha = jnp.dot(ex[k] / den, e4, preferred_element_type=f32)
            term = alpha * nbs[k]
            pooled = term if pooled is None else pooled + term
        o = (jnp.dot(selfv, w3a_ref[hop], preferred_element_type=f32)
             + jnp.dot(pooled, w3b_ref[hop], preferred_element_type=f32))
        return jnp.maximum(o, 0.0)

    g1f = g1_ref[...].reshape(NB * N1, INDIM)
    g1w3 = g1w_ref[...]
    g2w3 = g2w_ref[...]

    h1 = agg(g1f,
             lambda k: g2w3[:, :, k * INDIM:(k + 1) * INDIM].reshape(NB * N1, INDIM),
             w1_ref[...], p1_ref[...], s1600, 0, N1)
    h0 = agg(e0,
             lambda k: g1w3[:, :, k * INDIM:(k + 1) * INDIM].reshape(NB * SEQ, INDIM),
             w0_ref[...], p0_ref[...], s160, 0, SEQ)
    pt = pt_ref[...]
    h1_3 = h1.reshape(NB, N1, INDIM)
    nbf = jnp.stack([jnp.dot(pt, h1_3[b], preferred_element_type=f32)
                     for b in range(NB)])
    fin = agg(h0,
              lambda k: nbf[:, k * SEQ:(k + 1) * SEQ, :].reshape(NB * SEQ, INDIM),
              w0_ref[...], p0_ref[...], s160, 1, SEQ)

    q = (jnp.dot(fin, l1w_ref[...], preferred_element_type=f32) + l1b_ref[...]
         + jnp.dot(hf, l2w_ref[...], preferred_element_type=f32) + l2b_ref[...])
    alpha = jnp.dot(jax.nn.sigmoid(q), l3w_ref[...], preferred_element_type=f32)
    out_ref[...] = (alpha * hf + (1.0 - alpha) * fin).reshape(NB, SEQ, INDIM)


def _main(off, nbatch, h, shl, mask3, g1, g1w, g2w, w0r, p0r, w1r, p1r,
          wlcat, bcat, blockones, pt, e4,
          bdw1c, w1w, w1p, bdw2, bdw3a, bdw3b, l1w, l1b, l2w, l2b, l3w,
          seg, exp200, exp20):
    full = lambda shape: pl.BlockSpec(shape, lambda b: (0,) * len(shape))
    batch = lambda shape: pl.BlockSpec(
        (NB,) + shape, lambda b: (b + off,) + (0,) * len(shape))
    own = lambda shape: pl.BlockSpec(
        (NB,) + shape, lambda b: (b,) + (0,) * len(shape))
    return pl.pallas_call(
        _main_body,
        grid=(nbatch // NB,),
        in_specs=[
            batch((SEQ, INDIM)),
            batch((SEQ, INDIM)),
            batch((1, SEQ)),
            own((N1, INDIM)),
            own((SEQ, S * INDIM)),
            own((N1, S * INDIM)),
            batch((SEQ, S)),
            batch((SEQ, S * P)),
            batch((N1, S)),
            batch((N1, S * P)),
            full((INDIM, INDIM)),
            full((1, INDIM)),
            full((INDIM, INDIM)),
            full((N1, N1)),
            full((CH, INDIM)),
            full((2, INDIM, INDIM)),
            full((2, 1, INDIM)),
            full((2, P, INDIM)),
            full((2, INDIM, CH)),
            full((2, INDIM, INDIM)),
            full((2, INDIM, INDIM)),
            full((INDIM, INDIM)),
            full((1, INDIM)),
            full((INDIM, INDIM)),
            full((1, INDIM)),
            full((INDIM, 1)),
            full((NB, NB * SEQ)),
            full((NB * N1, NB)),
            full((NB * SEQ, NB)),
        ],
        out_specs=pl.BlockSpec((NB, SEQ, INDIM), lambda b: (b, 0, 0)),
        out_shape=jax.ShapeDtypeStruct((nbatch, SEQ, INDIM), jnp.float32),
    )(h, shl, mask3, g1, g1w, g2w, w0r, p0r, w1r, p1r,
      wlcat, bcat, blockones, pt, e4,
      bdw1c, w1w, w1p, bdw2, bdw3a, bdw3b, l1w, l1b, l2w, l2b, l3w,
      seg, exp200, exp20)


_PT_NP = np.zeros((N1, N1), np.float32)
_r = np.arange(N1)
_PT_NP[_r, (_r % SEQ) * S + (_r // SEQ)] = 1.0

_SEG_NP = np.zeros((NB, NB * SEQ), np.float32)
_SEG_NP[np.arange(NB * SEQ) // SEQ, np.arange(NB * SEQ)] = 1.0
_EXP200_NP = np.zeros((NB * N1, NB), np.float32)
_EXP200_NP[np.arange(NB * N1), np.arange(NB * N1) // N1] = 1.0
_EXP20_NP = np.zeros((NB * SEQ, NB), np.float32)
_EXP20_NP[np.arange(NB * SEQ), np.arange(NB * SEQ) // SEQ] = 1.0


def kernel(h, item_neighbors_0_0, item_neighbors_1_0, item_neighbors_2_0,
           weight_neighbors_0_0, weight_neighbors_1_0, pos_neighbors_0_0,
           pos_neighbors_1_0, pos_before, pos_after, seq_hidden_local,
           mask_item, embedding, weight_list, bias_list, agg_W1, agg_W2,
           agg_W3, lin1_W, lin1_b, lin2_W, lin2_b, lin3_W):
    f32 = jnp.float32
    eye4 = jnp.eye(CH, dtype=f32)

    wlcat = jnp.concatenate([weight_list[j] for j in range(CH)], axis=1)
    bcat = jnp.concatenate([bias_list[j] for j in range(CH)], axis=1)
    blockones = jnp.kron(eye4, jnp.ones((CDIM, CDIM), f32))
    e4 = jnp.kron(eye4, jnp.ones((1, CDIM), f32))
    bdw1c = jnp.stack([jnp.kron(eye4, agg_W1[i, :CDIM]) for i in range(2)])
    w1w = jnp.stack([jnp.tile(agg_W1[i, CDIM:CDIM + 1], (1, CH)) for i in range(2)])
    w1p = jnp.stack([jnp.tile(agg_W1[i, CDIM + 1:], (1, CH)) for i in range(2)])
    bdw2 = jnp.stack([jnp.kron(eye4, agg_W2[i]) for i in range(2)])
    bdw3a = jnp.stack([jnp.kron(eye4, agg_W3[i, :CDIM]) for i in range(2)])
    bdw3b = jnp.stack([jnp.kron(eye4, agg_W3[i, CDIM:]) for i in range(2)])
    pt = jnp.asarray(_PT_NP)
    seg = jnp.asarray(_SEG_NP)
    exp200 = jnp.asarray(_EXP200_NP)
    exp20 = jnp.asarray(_EXP20_NP)

    i1_flat = item_neighbors_1_0.astype(jnp.int32).reshape(T1)
    i2_flat = item_neighbors_2_0.astype(jnp.int32).reshape(T2)
    w0r = weight_neighbors_0_0.reshape(B, SEQ, S)
    w1r = weight_neighbors_1_0.reshape(B, N1, S)
    p0r = pos_neighbors_0_0.reshape(B, SEQ, S * P)
    p1r = pos_neighbors_1_0.reshape(B, N1, S * P)
    mask3 = mask_item.reshape(B, 1, SEQ)
    l1b = lin1_b.reshape(1, INDIM)
    l2b = lin2_b.reshape(1, INDIM)

    table = _route_table(embedding, wlcat, bcat, blockones)
    bh = B // 2
    t1h, t2h = T1 // 2, T2 // 2
    outs = []
    gh = []
    for half in range(2):
        i1h = lax.dynamic_slice_in_dim(i1_flat, half * t1h, t1h)
        i2h = lax.dynamic_slice_in_dim(i2_flat, half * t2h, t2h)
        gh.append(_sc_gather(table, i1h, i2h, t1h, t2h))
    for half in range(2):
        g1f, g2f = gh[half]
        g1 = g1f.reshape(bh, N1, INDIM)
        g1wide = g1f.reshape(bh, SEQ, S * INDIM)
        g2wide = g2f.reshape(bh, N1, S * INDIM)
        outs.append(_main(half * (bh // NB), bh,
                          h, seq_hidden_local, mask3, g1, g1wide, g2wide,
                          w0r, p0r, w1r, p1r,
                          wlcat, bcat, blockones, pt, e4,
                          bdw1c, w1w, w1p, bdw2, bdw3a, bdw3b,
                          lin1_W, l1b, lin2_W, l2b, lin3_W,
                          seg, exp200, exp20))
    return jnp.concatenate(outs, axis=0)


# speedup vs baseline: 3.2326x; 1.3847x over previous
import functools

import jax
import jax.numpy as jnp
import numpy as np
from jax import lax
from jax.experimental import pallas as pl
from jax.experimental.pallas import tpu as pltpu
from jax.experimental.pallas import tpu_sc as plsc

B = 128
SEQ = 20
S = 10
CH = 4
CDIM = 32
INDIM = 128
P = 16
VOCAB = 100000
N1 = SEQ * S
N2 = SEQ * S * S
T1 = B * N1
T2 = B * N2

ROWS_BLK = 2000


def _route_table_body(emb_ref, wl_ref, b_ref, ones_ref, out_ref):
    z = jnp.dot(emb_ref[...], wl_ref[...], preferred_element_type=jnp.float32)
    z = z + b_ref[...]
    nsq = jnp.dot(z * z, ones_ref[...], preferred_element_type=jnp.float32)
    out_ref[...] = z / jnp.maximum(jnp.sqrt(nsq), 1e-12)


def _route_table(emb, wlcat, bcat, blockones):
    grid = VOCAB // ROWS_BLK
    return pl.pallas_call(
        _route_table_body,
        grid=(grid,),
        in_specs=[
            pl.BlockSpec((ROWS_BLK, INDIM), lambda i: (i, 0)),
            pl.BlockSpec((INDIM, INDIM), lambda i: (0, 0)),
            pl.BlockSpec((1, INDIM), lambda i: (0, 0)),
            pl.BlockSpec((INDIM, INDIM), lambda i: (0, 0)),
        ],
        out_specs=pl.BlockSpec((ROWS_BLK, INDIM), lambda i: (i, 0)),
        out_shape=jax.ShapeDtypeStruct((VOCAB, INDIM), jnp.float32),
    )(emb, wlcat, bcat, blockones)


_NC = 2
_NS = 16
_NW = _NC * _NS
_WCH = 80
_NPC = 5
_WAVE = _WCH * _NPC


def _sc_gather(table, i1h, i2h, t1, t2):
    pw1, pw2 = t1 // _NW, t2 // _NW
    nw2 = pw2 // _WAVE
    mesh = plsc.VectorSubcoreMesh(core_axis_name="c", subcore_axis_name="s")

    @functools.partial(
        pl.kernel,
        mesh=mesh,
        out_type=(
            jax.ShapeDtypeStruct((t1, INDIM), jnp.float32),
            jax.ShapeDtypeStruct((t2, INDIM), jnp.float32),
        ),
        scratch_types=[
            pltpu.VMEM((pw1,), jnp.int32),
            pltpu.VMEM((pw2,), jnp.int32),
            pltpu.VMEM((_WAVE, INDIM), jnp.float32),
            pltpu.VMEM((_WAVE, INDIM), jnp.float32),
            pltpu.SemaphoreType.DMA,
            pltpu.SemaphoreType.DMA,
        ],
    )
    def k(table_hbm, i1_hbm, i2_hbm, out1_hbm, out2_hbm,
          idx1_v, idx2_v, ra, rb, sema, semb):
        wid = lax.axis_index("s") * _NC + lax.axis_index("c")
        b1 = wid * pw1
        b2 = wid * pw2
        pltpu.sync_copy(i1_hbm.at[pl.ds(b1, pw1)], idx1_v)
        pltpu.sync_copy(i2_hbm.at[pl.ds(b2, pw2)], idx2_v)

        def fire(idx_v, woff, buf, sem):
            for c in range(_NPC):
                pltpu.async_copy(
                    table_hbm.at[idx_v.at[pl.ds(woff + c * _WCH, _WCH)]],
                    buf.at[pl.ds(c * _WCH, _WCH)], sem)

        def drain(buf, sem):
            pltpu.make_async_copy(table_hbm.at[pl.ds(0, _WAVE)], buf, sem).wait()

        fire(idx1_v, 0, ra, sema)
        drain(ra, sema)
        pltpu.sync_copy(ra, out1_hbm.at[pl.ds(b1, pw1)])

        fire(idx2_v, 0, ra, sema)

        def body(i, carry):
            w0 = 2 * i
            fire(idx2_v, (w0 + 1) * _WAVE, rb, semb)
            drain(ra, sema)
            pltpu.sync_copy(ra, out2_hbm.at[pl.ds(b2 + w0 * _WAVE, _WAVE)])

            @pl.when(i < nw2 // 2 - 1)
            def _():
                fire(idx2_v, (w0 + 2) * _WAVE, ra, sema)

            drain(rb, semb)
            pltpu.sync_copy(rb, out2_hbm.at[pl.ds(b2 + (w0 + 1) * _WAVE, _WAVE)])
            return carry

        lax.fori_loop(0, nw2 // 2, body, 0)

    return k(table, i1h, i2h)


NB = 8
GRID3 = B // NB


def _main_body(h_ref, shl_ref, mask_ref, g1_ref, g2_ref,
               w0_ref, p0_ref, w1_ref, p1_ref,
               wlc_ref, bc_ref, bo_ref, pt_ref, e4_ref,
               w1c_ref, w1w_ref, w1p_ref, w2_ref, w3a_ref, w3b_ref,
               l1w_ref, l1b_ref, l2w_ref, l2b_ref, l3w_ref,
               seg_ref, exp200_ref, exp20_ref, out_ref):
    f32 = jnp.float32
    wlc = wlc_ref[...]
    bc = bc_ref[...]
    bo = bo_ref[...]

    def route(x):
        z = jnp.dot(x, wlc, preferred_element_type=f32) + bc
        nsq = jnp.dot(z * z, bo, preferred_element_type=f32)
        return z / jnp.maximum(jnp.sqrt(nsq), 1e-12)

    hf = h_ref[...].reshape(NB * SEQ, INDIM)
    e0 = route(hf)
    item = route(shl_ref[...].reshape(NB * SEQ, INDIM))
    msum = jnp.sum(mask_ref[...], axis=2)
    srow = jnp.dot(seg_ref[...], item, preferred_element_type=f32) / msum
    s1600 = jnp.dot(exp200_ref[...], srow, preferred_element_type=f32)
    s160 = jnp.dot(exp20_ref[...], srow, preferred_element_type=f32)
    e4 = e4_ref[...]

    def agg(selfv, getnb, wv3, pv3, sfull, hop, n):
        rows = NB * n
        nbs, logits = [], []
        for k in range(S):
            nb = getnb(k)
            m = nb * sfull
            wt = (wv3[:, :, k:k + 1].reshape(rows, 1) * w1w_ref[hop]
                  + jnp.dot(pv3[:, :, k * P:(k + 1) * P].reshape(rows, P),
                            w1p_ref[hop], preferred_element_type=f32))
            a = jnp.dot(m, w1c_ref[hop], preferred_element_type=f32) + wt
            a = jnp.where(a >= 0, a, 0.2 * a)
            nbs.append(nb)
            logits.append(jnp.dot(a, w2_ref[hop], preferred_element_type=f32))
        mx = logits[0]
        for k in range(1, S):
            mx = jnp.maximum(mx, logits[k])
        ex = [jnp.exp(l - mx) for l in logits]
        den = ex[0]
        for k in range(1, S):
            den = den + ex[k]
        pooled = None
        for k in range(S):
            alpha = jnp.dot(ex[k] / den, e4, preferred_element_type=f32)
            term = alpha * nbs[k]
            pooled = term if pooled is None else pooled + term
        o = (jnp.dot(selfv, w3a_ref[hop], preferred_element_type=f32)
             + jnp.dot(pooled, w3b_ref[hop], preferred_element_type=f32))
        return jnp.maximum(o, 0.0)

    g1_3 = g1_ref[...]
    g1f = g1_3.reshape(NB * N1, INDIM)
    g2r3 = g2_ref[...]
    pt = pt_ref[...]

    h1 = agg(g1f,
             lambda k: g2r3[:, k * N1:(k + 1) * N1, :].reshape(NB * N1, INDIM),
             w1_ref[...], p1_ref[...], s1600, 0, N1)
    g1p = jnp.stack([jnp.dot(pt, g1_3[b], preferred_element_type=f32)
                     for b in range(NB)])
    h0 = agg(e0,
             lambda k: g1p[:, k * SEQ:(k + 1) * SEQ, :].reshape(NB * SEQ, INDIM),
             w0_ref[...], p0_ref[...], s160, 0, SEQ)
    h1_3 = h1.reshape(NB, N1, INDIM)
    nbf = jnp.stack([jnp.dot(pt, h1_3[b], preferred_element_type=f32)
                     for b in range(NB)])
    fin = agg(h0,
              lambda k: nbf[:, k * SEQ:(k + 1) * SEQ, :].reshape(NB * SEQ, INDIM),
              w0_ref[...], p0_ref[...], s160, 1, SEQ)

    q = (jnp.dot(fin, l1w_ref[...], preferred_element_type=f32) + l1b_ref[...]
         + jnp.dot(hf, l2w_ref[...], preferred_element_type=f32) + l2b_ref[...])
    alpha = jnp.dot(jax.nn.sigmoid(q), l3w_ref[...], preferred_element_type=f32)
    out_ref[...] = (alpha * hf + (1.0 - alpha) * fin).reshape(NB, SEQ, INDIM)


def _main(off, nbatch, h, shl, mask3, g1, g2, w0r, p0r, w1r, p1r,
          wlcat, bcat, blockones, pt, e4,
          bdw1c, w1w, w1p, bdw2, bdw3a, bdw3b, l1w, l1b, l2w, l2b, l3w,
          seg, exp200, exp20):
    full = lambda shape: pl.BlockSpec(shape, lambda b: (0,) * len(shape))
    batch = lambda shape: pl.BlockSpec(
        (NB,) + shape, lambda b: (b + off,) + (0,) * len(shape))
    own = lambda shape: pl.BlockSpec(
        (NB,) + shape, lambda b: (b,) + (0,) * len(shape))
    return pl.pallas_call(
        _main_body,
        grid=(nbatch // NB,),
        in_specs=[
            batch((SEQ, INDIM)),
            batch((SEQ, INDIM)),
            batch((1, SEQ)),
            own((N1, INDIM)),
            own((N2, INDIM)),
            batch((SEQ, S)),
            batch((SEQ, S * P)),
            batch((N1, S)),
            batch((N1, S * P)),
            full((INDIM, INDIM)),
            full((1, INDIM)),
            full((INDIM, INDIM)),
            full((N1, N1)),
            full((CH, INDIM)),
            full((2, INDIM, INDIM)),
            full((2, 1, INDIM)),
            full((2, P, INDIM)),
            full((2, INDIM, CH)),
            full((2, INDIM, INDIM)),
            full((2, INDIM, INDIM)),
            full((INDIM, INDIM)),
            full((1, INDIM)),
            full((INDIM, INDIM)),
            full((1, INDIM)),
            full((INDIM, 1)),
            full((NB, NB * SEQ)),
            full((NB * N1, NB)),
            full((NB * SEQ, NB)),
        ],
        out_specs=pl.BlockSpec((NB, SEQ, INDIM), lambda b: (b, 0, 0)),
        out_shape=jax.ShapeDtypeStruct((nbatch, SEQ, INDIM), jnp.float32),
    )(h, shl, mask3, g1, g2, w0r, p0r, w1r, p1r,
      wlcat, bcat, blockones, pt, e4,
      bdw1c, w1w, w1p, bdw2, bdw3a, bdw3b, l1w, l1b, l2w, l2b, l3w,
      seg, exp200, exp20)


_PT_NP = np.zeros((N1, N1), np.float32)
_r = np.arange(N1)
_PT_NP[_r, (_r % SEQ) * S + (_r // SEQ)] = 1.0

_SEG_NP = np.zeros((NB, NB * SEQ), np.float32)
_SEG_NP[np.arange(NB * SEQ) // SEQ, np.arange(NB * SEQ)] = 1.0
_EXP200_NP = np.zeros((NB * N1, NB), np.float32)
_EXP200_NP[np.arange(NB * N1), np.arange(NB * N1) // N1] = 1.0
_EXP20_NP = np.zeros((NB * SEQ, NB), np.float32)
_EXP20_NP[np.arange(NB * SEQ), np.arange(NB * SEQ) // SEQ] = 1.0


def kernel(h, item_neighbors_0_0, item_neighbors_1_0, item_neighbors_2_0,
           weight_neighbors_0_0, weight_neighbors_1_0, pos_neighbors_0_0,
           pos_neighbors_1_0, pos_before, pos_after, seq_hidden_local,
           mask_item, embedding, weight_list, bias_list, agg_W1, agg_W2,
           agg_W3, lin1_W, lin1_b, lin2_W, lin2_b, lin3_W):
    f32 = jnp.float32
    eye4 = jnp.eye(CH, dtype=f32)

    wlcat = jnp.concatenate([weight_list[j] for j in range(CH)], axis=1)
    bcat = jnp.concatenate([bias_list[j] for j in range(CH)], axis=1)
    blockones = jnp.kron(eye4, jnp.ones((CDIM, CDIM), f32))
    e4 = jnp.kron(eye4, jnp.ones((1, CDIM), f32))
    bdw1c = jnp.stack([jnp.kron(eye4, agg_W1[i, :CDIM]) for i in range(2)])
    w1w = jnp.stack([jnp.tile(agg_W1[i, CDIM:CDIM + 1], (1, CH)) for i in range(2)])
    w1p = jnp.stack([jnp.tile(agg_W1[i, CDIM + 1:], (1, CH)) for i in range(2)])
    bdw2 = jnp.stack([jnp.kron(eye4, agg_W2[i]) for i in range(2)])
    bdw3a = jnp.stack([jnp.kron(eye4, agg_W3[i, :CDIM]) for i in range(2)])
    bdw3b = jnp.stack([jnp.kron(eye4, agg_W3[i, CDIM:]) for i in range(2)])
    pt = jnp.asarray(_PT_NP)
    seg = jnp.asarray(_SEG_NP)
    exp200 = jnp.asarray(_EXP200_NP)
    exp20 = jnp.asarray(_EXP20_NP)

    i1_flat = item_neighbors_1_0.astype(jnp.int32).reshape(T1)
    i2_flat = (item_neighbors_2_0.astype(jnp.int32)
               .reshape(B, N1, S).transpose(0, 2, 1).reshape(T2))
    w0r = weight_neighbors_0_0.reshape(B, SEQ, S)
    w1r = weight_neighbors_1_0.reshape(B, N1, S)
    p0r = pos_neighbors_0_0.reshape(B, SEQ, S * P)
    p1r = pos_neighbors_1_0.reshape(B, N1, S * P)
    mask3 = mask_item.reshape(B, 1, SEQ)
    l1b = lin1_b.reshape(1, INDIM)
    l2b = lin2_b.reshape(1, INDIM)

    table = _route_table(embedding, wlcat, bcat, blockones)
    bh = B // 2
    t1h, t2h = T1 // 2, T2 // 2
    outs = []
    gh = []
    for half in range(2):
        i1h = lax.dynamic_slice_in_dim(i1_flat, half * t1h, t1h)
        i2h = lax.dynamic_slice_in_dim(i2_flat, half * t2h, t2h)
        gh.append(_sc_gather(table, i1h, i2h, t1h, t2h))
    for half in range(2):
        g1f, g2f = gh[half]
        g1 = g1f.reshape(bh, N1, INDIM)
        g2 = g2f.reshape(bh, N2, INDIM)
        outs.append(_main(half * (bh // NB), bh,
                          h, seq_hidden_local, mask3, g1, g2,
                          w0r, p0r, w1r, p1r,
                          wlcat, bcat, blockones, pt, e4,
                          bdw1c, w1w, w1p, bdw2, bdw3a, bdw3b,
                          lin1_W, l1b, lin2_W, l2b, lin3_W,
                          seg, exp200, exp20))
    return jnp.concatenate(outs, axis=0)


# speedup vs baseline: 3.3733x; 1.0435x over previous
import functools

import jax
import jax.numpy as jnp
import numpy as np
from jax import lax
from jax.experimental import pallas as pl
from jax.experimental.pallas import tpu as pltpu
from jax.experimental.pallas import tpu_sc as plsc

B = 128
SEQ = 20
S = 10
CH = 4
CDIM = 32
INDIM = 128
P = 16
VOCAB = 100000
N1 = SEQ * S
N2 = SEQ * S * S
T1 = B * N1
T2 = B * N2

ROWS_BLK = 2000


def _route_table_body(emb_ref, wl_ref, b_ref, ones_ref, out_ref):
    z = jnp.dot(emb_ref[...], wl_ref[...], preferred_element_type=jnp.float32)
    z = z + b_ref[...]
    nsq = jnp.dot(z * z, ones_ref[...], preferred_element_type=jnp.float32)
    out_ref[...] = z / jnp.maximum(jnp.sqrt(nsq), 1e-12)


def _route_table(emb, wlcat, bcat, blockones):
    grid = VOCAB // ROWS_BLK
    return pl.pallas_call(
        _route_table_body,
        grid=(grid,),
        in_specs=[
            pl.BlockSpec((ROWS_BLK, INDIM), lambda i: (i, 0)),
            pl.BlockSpec((INDIM, INDIM), lambda i: (0, 0)),
            pl.BlockSpec((1, INDIM), lambda i: (0, 0)),
            pl.BlockSpec((INDIM, INDIM), lambda i: (0, 0)),
        ],
        out_specs=pl.BlockSpec((ROWS_BLK, INDIM), lambda i: (i, 0)),
        out_shape=jax.ShapeDtypeStruct((VOCAB, INDIM), jnp.float32),
    )(emb, wlcat, bcat, blockones)


_NC = 2
_NS = 16
_NW = _NC * _NS
_WCH = 80
_NPC = 5
_WAVE = _WCH * _NPC


def _sc_gather(table, i1h, i2h, t1, t2):
    pw1, pw2 = t1 // _NW, t2 // _NW
    nw2 = pw2 // _WAVE
    mesh = plsc.VectorSubcoreMesh(core_axis_name="c", subcore_axis_name="s")

    @functools.partial(
        pl.kernel,
        mesh=mesh,
        out_type=(
            jax.ShapeDtypeStruct((t1, INDIM), jnp.float32),
            jax.ShapeDtypeStruct((t2, INDIM), jnp.float32),
        ),
        scratch_types=[
            pltpu.VMEM((pw1,), jnp.int32),
            pltpu.VMEM((pw2,), jnp.int32),
            pltpu.VMEM((_WAVE, INDIM), jnp.float32),
            pltpu.VMEM((_WAVE, INDIM), jnp.float32),
            pltpu.SemaphoreType.DMA,
            pltpu.SemaphoreType.DMA,
        ],
    )
    def k(table_hbm, i1_hbm, i2_hbm, out1_hbm, out2_hbm,
          idx1_v, idx2_v, ra, rb, sema, semb):
        wid = lax.axis_index("s") * _NC + lax.axis_index("c")
        b1 = wid * pw1
        b2 = wid * pw2
        pltpu.sync_copy(i1_hbm.at[pl.ds(b1, pw1)], idx1_v)
        pltpu.sync_copy(i2_hbm.at[pl.ds(b2, pw2)], idx2_v)

        def fire(idx_v, woff, buf, sem):
            for c in range(_NPC):
                pltpu.async_copy(
                    table_hbm.at[idx_v.at[pl.ds(woff + c * _WCH, _WCH)]],
                    buf.at[pl.ds(c * _WCH, _WCH)], sem)

        def drain(buf, sem):
            pltpu.make_async_copy(table_hbm.at[pl.ds(0, _WAVE)], buf, sem).wait()

        fire(idx1_v, 0, ra, sema)
        drain(ra, sema)
        pltpu.sync_copy(ra, out1_hbm.at[pl.ds(b1, pw1)])

        fire(idx2_v, 0, ra, sema)

        def body(i, carry):
            w0 = 2 * i
            fire(idx2_v, (w0 + 1) * _WAVE, rb, semb)
            drain(ra, sema)
            pltpu.sync_copy(ra, out2_hbm.at[pl.ds(b2 + w0 * _WAVE, _WAVE)])

            @pl.when(i < nw2 // 2 - 1)
            def _():
                fire(idx2_v, (w0 + 2) * _WAVE, ra, sema)

            drain(rb, semb)
            pltpu.sync_copy(rb, out2_hbm.at[pl.ds(b2 + (w0 + 1) * _WAVE, _WAVE)])
            return carry

        lax.fori_loop(0, nw2 // 2, body, 0)

    return k(table, i1h, i2h)


NB = 8
GRID3 = B // NB


def _main_body(h_ref, shl_ref, mask_ref, g1_ref, g2_ref,
               w0_ref, p0_ref, w1_ref, p1_ref,
               wlc_ref, bc_ref, bo_ref, pt_ref, e4_ref,
               w1c_ref, w1w_ref, w1p_ref, w2_ref, w3a_ref, w3b_ref,
               l1w_ref, l1b_ref, l2w_ref, l2b_ref, l3w_ref,
               seg_ref, exp200_ref, exp20_ref, out_ref):
    f32 = jnp.float32
    wlc = wlc_ref[...]
    bc = bc_ref[...]
    bo = bo_ref[...]

    def route(x):
        z = jnp.dot(x, wlc, preferred_element_type=f32) + bc
        nsq = jnp.dot(z * z, bo, preferred_element_type=f32)
        return z / jnp.maximum(jnp.sqrt(nsq), 1e-12)

    hf = h_ref[...].reshape(NB * SEQ, INDIM)
    e0 = route(hf)
    item = route(shl_ref[...].reshape(NB * SEQ, INDIM))
    msum = jnp.sum(mask_ref[...], axis=2)
    srow = jnp.dot(seg_ref[...], item, preferred_element_type=f32) / msum
    s1600 = jnp.dot(exp200_ref[...], srow, preferred_element_type=f32)
    s160 = jnp.dot(exp20_ref[...], srow, preferred_element_type=f32)
    e4 = e4_ref[...]

    bf16 = jnp.bfloat16
    w1c_b = [w1c_ref[i].astype(bf16) for i in range(2)]
    w3a_b = [w3a_ref[i].astype(bf16) for i in range(2)]
    w3b_b = [w3b_ref[i].astype(bf16) for i in range(2)]

    def agg(selfv, getnb, wv3, pv3, sfull, hop, n):
        rows = NB * n
        nbs, logits = [], []
        for k in range(S):
            nb = getnb(k)
            m = nb * sfull
            wt = (wv3[:, :, k:k + 1].reshape(rows, 1) * w1w_ref[hop]
                  + jnp.dot(pv3[:, :, k * P:(k + 1) * P].reshape(rows, P),
                            w1p_ref[hop], preferred_element_type=f32))
            a = jnp.dot(m.astype(bf16), w1c_b[hop],
                        preferred_element_type=f32) + wt
            a = jnp.where(a >= 0, a, 0.2 * a)
            nbs.append(nb)
            logits.append(jnp.dot(a, w2_ref[hop], preferred_element_type=f32))
        mx = logits[0]
        for k in range(1, S):
            mx = jnp.maximum(mx, logits[k])
        ex = [jnp.exp(l - mx) for l in logits]
        den = ex[0]
        for k in range(1, S):
            den = den + ex[k]
        pooled = None
        for k in range(S):
            alpha = jnp.dot(ex[k] / den, e4, preferred_element_type=f32)
            term = alpha * nbs[k]
            pooled = term if pooled is None else pooled + term
        o = (jnp.dot(selfv.astype(bf16), w3a_b[hop], preferred_element_type=f32)
             + jnp.dot(pooled.astype(bf16), w3b_b[hop], preferred_element_type=f32))
        return jnp.maximum(o, 0.0)

    g1_3 = g1_ref[...].astype(f32)
    g1f = g1_3.reshape(NB * N1, INDIM)
    g2r3 = g2_ref[...]
    pt = pt_ref[...]

    h1 = agg(g1f,
             lambda k: g2r3[:, k * N1:(k + 1) * N1, :]
             .reshape(NB * N1, INDIM).astype(f32),
             w1_ref[...], p1_ref[...], s1600, 0, N1)
    g1p = jnp.stack([jnp.dot(pt, g1_3[b], preferred_element_type=f32)
                     for b in range(NB)])
    h0 = agg(e0,
             lambda k: g1p[:, k * SEQ:(k + 1) * SEQ, :].reshape(NB * SEQ, INDIM),
             w0_ref[...], p0_ref[...], s160, 0, SEQ)
    h1_3 = h1.reshape(NB, N1, INDIM)
    nbf = jnp.stack([jnp.dot(pt, h1_3[b], preferred_element_type=f32)
                     for b in range(NB)])
    fin = agg(h0,
              lambda k: nbf[:, k * SEQ:(k + 1) * SEQ, :].reshape(NB * SEQ, INDIM),
              w0_ref[...], p0_ref[...], s160, 1, SEQ)

    q = (jnp.dot(fin, l1w_ref[...], preferred_element_type=f32) + l1b_ref[...]
         + jnp.dot(hf, l2w_ref[...], preferred_element_type=f32) + l2b_ref[...])
    alpha = jnp.dot(jax.nn.sigmoid(q), l3w_ref[...], preferred_element_type=f32)
    out_ref[...] = (alpha * hf + (1.0 - alpha) * fin).reshape(NB, SEQ, INDIM)


def _main(off, nbatch, h, shl, mask3, g1, g2, w0r, p0r, w1r, p1r,
          wlcat, bcat, blockones, pt, e4,
          bdw1c, w1w, w1p, bdw2, bdw3a, bdw3b, l1w, l1b, l2w, l2b, l3w,
          seg, exp200, exp20):
    full = lambda shape: pl.BlockSpec(shape, lambda b: (0,) * len(shape))
    batch = lambda shape: pl.BlockSpec(
        (NB,) + shape, lambda b: (b + off,) + (0,) * len(shape))
    own = lambda shape: pl.BlockSpec(
        (NB,) + shape, lambda b: (b,) + (0,) * len(shape))
    return pl.pallas_call(
        _main_body,
        grid=(nbatch // NB,),
        in_specs=[
            batch((SEQ, INDIM)),
            batch((SEQ, INDIM)),
            batch((1, SEQ)),
            own((N1, INDIM)),
            own((N2, INDIM)),
            batch((SEQ, S)),
            batch((SEQ, S * P)),
            batch((N1, S)),
            batch((N1, S * P)),
            full((INDIM, INDIM)),
            full((1, INDIM)),
            full((INDIM, INDIM)),
            full((N1, N1)),
            full((CH, INDIM)),
            full((2, INDIM, INDIM)),
            full((2, 1, INDIM)),
            full((2, P, INDIM)),
            full((2, INDIM, CH)),
            full((2, INDIM, INDIM)),
            full((2, INDIM, INDIM)),
            full((INDIM, INDIM)),
            full((1, INDIM)),
            full((INDIM, INDIM)),
            full((1, INDIM)),
            full((INDIM, 1)),
            full((NB, NB * SEQ)),
            full((NB * N1, NB)),
            full((NB * SEQ, NB)),
        ],
        out_specs=pl.BlockSpec((NB, SEQ, INDIM), lambda b: (b, 0, 0)),
        out_shape=jax.ShapeDtypeStruct((nbatch, SEQ, INDIM), jnp.float32),
    )(h, shl, mask3, g1, g2, w0r, p0r, w1r, p1r,
      wlcat, bcat, blockones, pt, e4,
      bdw1c, w1w, w1p, bdw2, bdw3a, bdw3b, l1w, l1b, l2w, l2b, l3w,
      seg, exp200, exp20)


_PT_NP = np.zeros((N1, N1), np.float32)
_r = np.arange(N1)
_PT_NP[_r, (_r % SEQ) * S + (_r // SEQ)] = 1.0

_SEG_NP = np.zeros((NB, NB * SEQ), np.float32)
_SEG_NP[np.arange(NB * SEQ) // SEQ, np.arange(NB * SEQ)] = 1.0
_EXP200_NP = np.zeros((NB * N1, NB), np.float32)
_EXP200_NP[np.arange(NB * N1), np.arange(NB * N1) // N1] = 1.0
_EXP20_NP = np.zeros((NB * SEQ, NB), np.float32)
_EXP20_NP[np.arange(NB * SEQ), np.arange(NB * SEQ) // SEQ] = 1.0


def kernel(h, item_neighbors_0_0, item_neighbors_1_0, item_neighbors_2_0,
           weight_neighbors_0_0, weight_neighbors_1_0, pos_neighbors_0_0,
           pos_neighbors_1_0, pos_before, pos_after, seq_hidden_local,
           mask_item, embedding, weight_list, bias_list, agg_W1, agg_W2,
           agg_W3, lin1_W, lin1_b, lin2_W, lin2_b, lin3_W):
    f32 = jnp.float32
    eye4 = jnp.eye(CH, dtype=f32)

    wlcat = jnp.concatenate([weight_list[j] for j in range(CH)], axis=1)
    bcat = jnp.concatenate([bias_list[j] for j in range(CH)], axis=1)
    blockones = jnp.kron(eye4, jnp.ones((CDIM, CDIM), f32))
    e4 = jnp.kron(eye4, jnp.ones((1, CDIM), f32))
    bdw1c = jnp.stack([jnp.kron(eye4, agg_W1[i, :CDIM]) for i in range(2)])
    w1w = jnp.stack([jnp.tile(agg_W1[i, CDIM:CDIM + 1], (1, CH)) for i in range(2)])
    w1p = jnp.stack([jnp.tile(agg_W1[i, CDIM + 1:], (1, CH)) for i in range(2)])
    bdw2 = jnp.stack([jnp.kron(eye4, agg_W2[i]) for i in range(2)])
    bdw3a = jnp.stack([jnp.kron(eye4, agg_W3[i, :CDIM]) for i in range(2)])
    bdw3b = jnp.stack([jnp.kron(eye4, agg_W3[i, CDIM:]) for i in range(2)])
    pt = jnp.asarray(_PT_NP)
    seg = jnp.asarray(_SEG_NP)
    exp200 = jnp.asarray(_EXP200_NP)
    exp20 = jnp.asarray(_EXP20_NP)

    i1_flat = item_neighbors_1_0.astype(jnp.int32).reshape(T1)
    i2_flat = (item_neighbors_2_0.astype(jnp.int32)
               .reshape(B, N1, S).transpose(0, 2, 1).reshape(T2))
    w0r = weight_neighbors_0_0.reshape(B, SEQ, S)
    w1r = weight_neighbors_1_0.reshape(B, N1, S)
    p0r = pos_neighbors_0_0.reshape(B, SEQ, S * P)
    p1r = pos_neighbors_1_0.reshape(B, N1, S * P)
    mask3 = mask_item.reshape(B, 1, SEQ)
    l1b = lin1_b.reshape(1, INDIM)
    l2b = lin2_b.reshape(1, INDIM)

    table = _route_table(embedding, wlcat, bcat, blockones)
    bh = B // 2
    t1h, t2h = T1 // 2, T2 // 2
    outs = []
    gh = []
    for half in range(2):
        i1h = lax.dynamic_slice_in_dim(i1_flat, half * t1h, t1h)
        i2h = lax.dynamic_slice_in_dim(i2_flat, half * t2h, t2h)
        gh.append(_sc_gather(table, i1h, i2h, t1h, t2h))
    for half in range(2):
        g1f, g2f = gh[half]
        g1 = g1f.reshape(bh, N1, INDIM)
        g2 = g2f.reshape(bh, N2, INDIM)
        outs.append(_main(half * (bh // NB), bh,
                          h, seq_hidden_local, mask3, g1, g2,
                          w0r, p0r, w1r, p1r,
                          wlcat, bcat, blockones, pt, e4,
                          bdw1c, w1w, w1p, bdw2, bdw3a, bdw3b,
                          lin1_W, l1b, lin2_W, l2b, lin3_W,
                          seg, exp200, exp20))
    return jnp.concatenate(outs, axis=0)
